# Initial kernel scaffold; baseline (speedup 1.0000x reference)
import jax
import jax.numpy as jnp
from jax.experimental import pallas as pl


def kernel(x, edge_index, edge_weight, shared_weight, output_bias):
    raise NotImplementedError("write your pallas kernel here")


# speedup vs baseline: 2.9749x; 2.9749x over previous
import functools

import jax
import jax.numpy as jnp
from jax import lax
from jax.experimental import pallas as pl
from jax.experimental.pallas import tpu as pltpu
from jax.experimental.pallas import tpu_sc as plsc

N = 10000
D = 128
E = 320000
C = 128
NC = 2
NS = 16
NW = NC * NS
CHUNKS_TOTAL = -(-E // (C * NW))
E_PAD = CHUNKS_TOTAL * C * NW
PER_W = E_PAD // NW
CHUNKS = PER_W // C
N_PAD = 10240
RPT = N_PAD // NS


def _spmm_sc(h, src, dst, w):
    mesh = plsc.VectorSubcoreMesh(core_axis_name="c", subcore_axis_name="s")

    @functools.partial(
        pl.kernel,
        out_type=jax.ShapeDtypeStruct((NC, N_PAD, D), jnp.float32),
        mesh=mesh,
        scratch_types=[
            pltpu.VMEM((C,), jnp.int32),
            pltpu.VMEM((C,), jnp.int32),
            pltpu.VMEM((C + 16,), jnp.float32),
            pltpu.VMEM((C, D), jnp.float32),
            pltpu.VMEM_SHARED((N_PAD, D), jnp.float32),
            pltpu.SemaphoreType.DMA,
        ],
    )
    def k(h_hbm, src_hbm, dst_hbm, w_hbm, out_hbm,
          idx_v, dst_v, w_v, rows_v, accum, sem):
        cid = lax.axis_index("c")
        sid = lax.axis_index("s")
        wid = sid * NC + cid

        def zrow(r, _):
            for j in range(D // 16):
                rows_v[r, pl.ds(j * 16, 16)] = jnp.zeros((16,), jnp.float32)
            return 0
        lax.fori_loop(0, C, zrow, 0)
        base = sid * RPT
        for kblk in range(RPT // C):
            pltpu.sync_copy(rows_v, accum.at[pl.ds(base + kblk * C, C)])
        plsc.subcore_barrier()

        ebase = wid * PER_W

        def chunk_body(ci, _):
            off = ebase + ci * C
            pltpu.sync_copy(src_hbm.at[pl.ds(off, C)], idx_v)
            pltpu.sync_copy(dst_hbm.at[pl.ds(off, C)], dst_v)
            pltpu.sync_copy(w_hbm.at[pl.ds(off, C)], w_v.at[pl.ds(0, C)])
            pltpu.async_copy(h_hbm.at[idx_v], rows_v, sem).wait()

            def row_body(r, _):
                ws = w_v[pl.ds(r, 16)][0]
                for j in range(D // 16):
                    sl = pl.ds(j * 16, 16)
                    rows_v[r, sl] = rows_v[r, sl] * ws
                return 0
            lax.fori_loop(0, C, row_body, 0)

            pltpu.sync_copy(rows_v, accum.at[dst_v], add=True)
            return 0

        lax.fori_loop(0, CHUNKS, chunk_body, 0)
        plsc.subcore_barrier()

        pltpu.sync_copy(accum.at[pl.ds(base, RPT)],
                        out_hbm.at[cid, pl.ds(base, RPT)])

    return k(h, src, dst, w)


R_BLK = 1000


def _fuse1(x, p0, p1, w0, w1):
    def body(x_b, p0_b, p1_b, w0_b, w1_b, h1_b, acc_b):
        h1 = p0_b[...] + p1_b[...]
        h1_b[...] = h1
        acc_b[...] = (
            jnp.dot(x_b[...], w0_b[...], preferred_element_type=jnp.float32)
            + jnp.dot(h1, w1_b[...], preferred_element_type=jnp.float32)
        )

    row_spec = pl.BlockSpec((R_BLK, D), lambda i: (i, 0))
    w_spec = pl.BlockSpec((D, D), lambda i: (0, 0))
    return pl.pallas_call(
        body,
        grid=(N // R_BLK,),
        in_specs=[row_spec, row_spec, row_spec, w_spec, w_spec],
        out_specs=[row_spec, row_spec],

        out_shape=[
            jax.ShapeDtypeStruct((N, D), jnp.float32),
            jax.ShapeDtypeStruct((N, D), jnp.float32),
        ],
    )(x, p0, p1, w0, w1)


def _fuse2(acc, q0, q1, w2, bias):
    def body(acc_b, q0_b, q1_b, w2_b, b_b, out_b):
        h2 = q0_b[...] + q1_b[...]
        out_b[...] = (
            acc_b[...]
            + jnp.dot(h2, w2_b[...], preferred_element_type=jnp.float32)
            + b_b[...]
        )

    row_spec = pl.BlockSpec((R_BLK, D), lambda i: (i, 0))
    w_spec = pl.BlockSpec((D, D), lambda i: (0, 0))
    b_spec = pl.BlockSpec((1, D), lambda i: (0, 0))
    return pl.pallas_call(
        body,
        grid=(N // R_BLK,),
        in_specs=[row_spec, row_spec, row_spec, w_spec, b_spec],
        out_specs=row_spec,
        out_shape=jax.ShapeDtypeStruct((N, D), jnp.float32),
    )(acc, q0, q1, w2, bias)


def kernel(x, edge_index, edge_weight, shared_weight, output_bias):
    src = edge_index[1].astype(jnp.int32)
    dst = edge_index[0].astype(jnp.int32)
    w = edge_weight.astype(jnp.float32)
    pad = E_PAD - E
    src = jnp.concatenate([src, jnp.zeros((pad,), jnp.int32)])
    dst = jnp.concatenate([dst, jnp.zeros((pad,), jnp.int32)])
    w = jnp.concatenate([w, jnp.zeros((pad,), jnp.float32)])

    w0 = shared_weight[:D]
    w1 = shared_weight[D:2 * D]
    w2 = shared_weight[2 * D:]
    bias = output_bias.reshape(1, D)

    p = _spmm_sc(x, src, dst, w)
    h1, acc = _fuse1(x, p[0], p[1], w0, w1)
    q = _spmm_sc(h1, src, dst, w)
    return _fuse2(acc, q[0], q[1], w2, bias)
